# baseline (speedup 1.0000x reference)
import jax
import jax.numpy as jnp
from jax import lax
from jax.experimental import pallas as pl
from jax.experimental.pallas import tpu as pltpu
from jax.experimental.pallas import tpu_sc as plsc

_B, _L, _D, _M = 1024, 50, 64, 10
_BL = _B * _L
_NW = 32
_NS = _BL // _NW
_NI = _NS * _M
_CQ = 80
_CI = 128
_PAD = 0


def _sc_body(sample_ref, map_query_ref, map_items_flat_ref,
             query_table_ref, item_table_ref,
             out_q_ref, out_i_ref, out_m_ref,
             sids_v, qids_v, eidx_v, iids_v, mask_v, qrows_v, irows_v, sem):
  wid = lax.axis_index("s") * 2 + lax.axis_index("c")
  base = wid * _NS
  iota = lax.iota(jnp.int32, 16)
  ten = jnp.full((16,), _M, jnp.int32)
  zrow = jnp.zeros((16,), jnp.float32)
  ones = jnp.full((16,), 1, jnp.int32)
  zeros_i = jnp.full((16,), 0, jnp.int32)

  pltpu.sync_copy(sample_ref.at[pl.ds(base, _NS)], sids_v)

  @pl.loop(0, _NS // _CQ)
  def _qid_gather(k):
    o = k * _CQ
    pltpu.async_copy(
        map_query_ref.at[sids_v.at[pl.ds(o, _CQ)]],
        qids_v.at[pl.ds(o, _CQ)], sem).wait()

  @pl.loop(0, _NI // 128)
  def _eidx(k):
    for g in range(8):
      j = k * 128 + g * 16 + iota
      srow = lax.div(j, ten)
      col = j - srow * _M
      sid = plsc.load_gather(sids_v, [srow])
      eidx_v[pl.ds(k * 128 + g * 16, 16)] = sid * _M + col

  @pl.loop(0, _NI // _CI)
  def _iid_gather(k):
    o = k * _CI
    pltpu.async_copy(
        map_items_flat_ref.at[eidx_v.at[pl.ds(o, _CI)]],
        iids_v.at[pl.ds(o, _CI)], sem).wait()

  @pl.loop(0, _NS // _CQ)
  def _q_rows(k):
    o = k * _CQ
    pltpu.async_copy(
        query_table_ref.at[qids_v.at[pl.ds(o, _CQ)]], qrows_v, sem).wait()
    for g in range(_CQ // 16):
      m = sids_v[pl.ds(o + g * 16, 16)] != _PAD

      @pl.when(jnp.logical_not(jnp.all(m)))
      def _fixup():
        rows = g * 16 + iota

        @pl.loop(0, _D)
        def _zero_col(c):
          ccol = jnp.full((16,), c, jnp.int32)
          plsc.store_scatter(qrows_v, [rows, ccol], zrow,
                             mask=jnp.logical_not(m))
    pltpu.sync_copy(qrows_v, out_q_ref.at[pl.ds(base + o, _CQ), :])

  @pl.loop(0, _NI // _CI)
  def _i_rows(k):
    o = k * _CI
    pltpu.async_copy(
        item_table_ref.at[iids_v.at[pl.ds(o, _CI)]], irows_v, sem).wait()
    for g in range(_CI // 16):
      j = o + g * 16 + iota
      srow = lax.div(j, ten)
      sid = plsc.load_gather(sids_v, [srow])
      iid = iids_v[pl.ds(o + g * 16, 16)]
      m_sess = sid != _PAD
      mask_v[pl.ds(o + g * 16, 16)] = jnp.where(
          m_sess & (iid != _PAD), ones, zeros_i)

      @pl.when(jnp.logical_not(jnp.all(m_sess)))
      def _fixup():
        rows = g * 16 + iota

        @pl.loop(0, _D)
        def _zero_col(c):
          ccol = jnp.full((16,), c, jnp.int32)
          plsc.store_scatter(irows_v, [rows, ccol], zrow,
                             mask=jnp.logical_not(m_sess))
    pltpu.sync_copy(irows_v, out_i_ref.at[pl.ds(base * _M + o, _CI), :])

  pltpu.sync_copy(mask_v, out_m_ref.at[pl.ds(base * _M, _NI)])


@jax.jit
def _sc_call(sample_flat, map_query, map_items_flat, query_table, item_table):
  mesh = plsc.VectorSubcoreMesh(core_axis_name="c", subcore_axis_name="s")
  f = pl.kernel(
      _sc_body,
      out_type=(
          jax.ShapeDtypeStruct((_BL, _D), jnp.float32),
          jax.ShapeDtypeStruct((_BL * _M, _D), jnp.float32),
          jax.ShapeDtypeStruct((_BL * _M,), jnp.int32),
      ),
      mesh=mesh,
      compiler_params=pltpu.CompilerParams(
          needs_layout_passes=False, use_tc_tiling_on_sc=False),
      scratch_types=[
          pltpu.VMEM((_NS,), jnp.int32),
          pltpu.VMEM((_NS,), jnp.int32),
          pltpu.VMEM((_NI,), jnp.int32),
          pltpu.VMEM((_NI,), jnp.int32),
          pltpu.VMEM((_NI,), jnp.int32),
          pltpu.VMEM((_CQ, _D), jnp.float32),
          pltpu.VMEM((_CI, _D), jnp.float32),
          pltpu.SemaphoreType.DMA,
      ],
  )
  return f(sample_flat, map_query, map_items_flat, query_table, item_table)


def kernel(sample, map_query, map_pos_items, query_table, item_table):
  out_q, out_i, out_m = _sc_call(
      sample.reshape(-1), map_query, map_pos_items.reshape(-1),
      query_table, item_table)
  return (
      out_q.reshape(_B, _L, _D),
      out_i.reshape(_B, _L, _M, _D),
      out_m.reshape(_B, _L, _M).astype(jnp.bool_),
  )


# speedup vs baseline: 1.1635x; 1.1635x over previous
import jax
import jax.numpy as jnp
from jax import lax
from jax.experimental import pallas as pl
from jax.experimental.pallas import tpu as pltpu
from jax.experimental.pallas import tpu_sc as plsc

_B, _L, _D, _M = 1024, 50, 64, 10
_BL = _B * _L
_NW = 32
_NS = _BL // _NW
_NI = _NS * _M
_CQ = 80
_NQC = _NS // _CQ
_CI = 128
_NIC = _NI // _CI
_PAD = 0


def _sc_body(sample_ref, map_query_ref, map_items_flat_ref,
             query_table_ref, item_table_ref,
             out_q_ref, out_i_ref, out_m_ref,
             sids_v, qids_v, eidx_v, iids_v, mask_v, pad_v,
             qb0, qb1, ib0, ib1, ib2, ib3,
             sem_q, sem_i, sem_m,
             g0, g1, g2, g3, w0, w1, w2, w3):
  wid = lax.axis_index("s") * 2 + lax.axis_index("c")
  base = wid * _NS
  iota = lax.iota(jnp.int32, 16)
  ten = jnp.full((16,), _M, jnp.int32)
  zrow = jnp.zeros((16,), jnp.float32)
  ones = jnp.full((16,), 1, jnp.int32)
  zeros_i = jnp.full((16,), 0, jnp.int32)
  qbufs = (qb0, qb1)
  ibufs = (ib0, ib1, ib2, ib3)
  gsems = (g0, g1, g2, g3)
  wsems = (w0, w1, w2, w3)

  pltpu.sync_copy(sample_ref.at[pl.ds(base, _NS)], sids_v)

  @pl.loop(0, 12)
  def _qid_fire(k):
    o = k * 128
    pltpu.async_copy(map_query_ref.at[sids_v.at[pl.ds(o, 128)]],
                     qids_v.at[pl.ds(o, 128)], sem_q)
  pltpu.async_copy(map_query_ref.at[sids_v.at[pl.ds(1536, 64)]],
                   qids_v.at[pl.ds(1536, 64)], sem_q)

  pad_v[...] = zeros_i

  @pl.loop(0, _NI // 128)
  def _eidx(k):
    for g in range(8):
      j = k * 128 + g * 16 + iota
      srow = lax.div(j, ten)
      col = j - srow * _M
      sid = plsc.load_gather(sids_v, [srow])
      eidx_v[pl.ds(k * 128 + g * 16, 16)] = sid * _M + col
      pad_v[...] = pad_v[...] | jnp.where(sid == _PAD, ones, zeros_i)

  has_pad = jnp.max(pad_v[...]) != 0

  @pl.loop(0, _NIC)
  def _iid_fire(k):
    o = k * _CI
    pltpu.async_copy(map_items_flat_ref.at[eidx_v.at[pl.ds(o, _CI)]],
                     iids_v.at[pl.ds(o, _CI)], sem_i)

  pltpu.make_async_copy(map_query_ref.at[pl.ds(0, _NS)], qids_v, sem_q).wait()

  pltpu.async_copy(query_table_ref.at[qids_v.at[pl.ds(0, _CQ)]], qb0, g0)
  pltpu.async_copy(query_table_ref.at[qids_v.at[pl.ds(_CQ, _CQ)]], qb1, g1)

  @pl.loop(0, _NQC // 2)
  def _q_ring(k):
    for b in range(2):
      c = k * 2 + b
      o = c * _CQ
      buf, gs, ws = qbufs[b], gsems[b], wsems[b]
      pltpu.make_async_copy(
          query_table_ref.at[qids_v.at[pl.ds(0, _CQ)]], buf, gs).wait()

      @pl.when(has_pad)
      def _fixup():
        for gq in range(_CQ // 16):
          m = sids_v[pl.ds(o + gq * 16, 16)] == _PAD

          @pl.when(jnp.any(m))
          def _zero():
            rows = gq * 16 + iota

            @pl.loop(0, _D)
            def _zc(cc):
              ccol = jnp.full((16,), cc, jnp.int32)
              plsc.store_scatter(buf, [rows, ccol], zrow, mask=m)

      pltpu.async_copy(buf, out_q_ref.at[pl.ds(base + o, _CQ), :], ws)

      @pl.when(c < _NQC - 2)
      def _next():
        pltpu.make_async_copy(
            buf, out_q_ref.at[pl.ds(base, _CQ), :], ws).wait()
        pltpu.async_copy(
            query_table_ref.at[qids_v.at[pl.ds(o + 2 * _CQ, _CQ)]], buf, gs)

  pltpu.make_async_copy(qb0, out_q_ref.at[pl.ds(base, _CQ), :], w0).wait()
  pltpu.make_async_copy(qb1, out_q_ref.at[pl.ds(base, _CQ), :], w1).wait()

  pltpu.make_async_copy(
      map_items_flat_ref.at[pl.ds(0, _NI)], iids_v, sem_i).wait()

  @pl.loop(0, _NI // 16)
  def _mask(k):
    iid = iids_v[pl.ds(k * 16, 16)]
    mask_v[pl.ds(k * 16, 16)] = jnp.where(iid != _PAD, ones, zeros_i)

  @pl.when(has_pad)
  def _mask_fix():
    @pl.loop(0, _NS // 16)
    def _mfix(g):
      sid = sids_v[pl.ds(g * 16, 16)]
      m = sid == _PAD

      @pl.when(jnp.any(m))
      def _zero():
        srow = g * 16 + iota
        for mm in range(_M):
          plsc.store_scatter(mask_v, [srow * _M + mm], zeros_i, mask=m)

  pltpu.async_copy(mask_v, out_m_ref.at[pl.ds(base * _M, _NI)], sem_m)

  for b in range(4):
    pltpu.async_copy(
        item_table_ref.at[iids_v.at[pl.ds(b * _CI, _CI)]], ibufs[b], gsems[b])

  @pl.loop(0, (_NIC + 3) // 4)
  def _i_ring(k):
    for b in range(4):
      c = k * 4 + b

      @pl.when(c < _NIC)
      def _chunk():
        o = c * _CI
        buf, gs, ws = ibufs[b], gsems[b], wsems[b]
        pltpu.make_async_copy(
            item_table_ref.at[iids_v.at[pl.ds(0, _CI)]], buf, gs).wait()

        @pl.when(has_pad)
        def _fixup():
          for gq in range(_CI // 16):
            j = o + gq * 16 + iota
            srow = lax.div(j, ten)
            sid = plsc.load_gather(sids_v, [srow])
            m = sid == _PAD

            @pl.when(jnp.any(m))
            def _zero():
              rows = gq * 16 + iota

              @pl.loop(0, _D)
              def _zc(cc):
                ccol = jnp.full((16,), cc, jnp.int32)
                plsc.store_scatter(buf, [rows, ccol], zrow, mask=m)

        pltpu.async_copy(buf, out_i_ref.at[pl.ds(base * _M + o, _CI), :], ws)

        @pl.when(c + 4 < _NIC)
        def _next():
          pltpu.make_async_copy(
              buf, out_i_ref.at[pl.ds(base * _M, _CI), :], ws).wait()
          pltpu.async_copy(
              item_table_ref.at[iids_v.at[pl.ds(o + 4 * _CI, _CI)]], buf, gs)

  for b in range(4):
    pltpu.make_async_copy(
        ibufs[b], out_i_ref.at[pl.ds(base * _M, _CI), :], wsems[b]).wait()

  pltpu.make_async_copy(
      mask_v, out_m_ref.at[pl.ds(base * _M, _NI)], sem_m).wait()


@jax.jit
def _sc_call(sample_flat, map_query, map_items_flat, query_table, item_table):
  mesh = plsc.VectorSubcoreMesh(core_axis_name="c", subcore_axis_name="s")
  f = pl.kernel(
      _sc_body,
      out_type=(
          jax.ShapeDtypeStruct((_BL, _D), jnp.float32),
          jax.ShapeDtypeStruct((_BL * _M, _D), jnp.float32),
          jax.ShapeDtypeStruct((_BL * _M,), jnp.int32),
      ),
      mesh=mesh,
      compiler_params=pltpu.CompilerParams(
          needs_layout_passes=False, use_tc_tiling_on_sc=False),
      scratch_types=[
          pltpu.VMEM((_NS,), jnp.int32),
          pltpu.VMEM((_NS,), jnp.int32),
          pltpu.VMEM((_NI,), jnp.int32),
          pltpu.VMEM((_NI,), jnp.int32),
          pltpu.VMEM((_NI,), jnp.int32),
          pltpu.VMEM((16,), jnp.int32),
          pltpu.VMEM((_CQ, _D), jnp.float32),
          pltpu.VMEM((_CQ, _D), jnp.float32),
          pltpu.VMEM((_CI, _D), jnp.float32),
          pltpu.VMEM((_CI, _D), jnp.float32),
          pltpu.VMEM((_CI, _D), jnp.float32),
          pltpu.VMEM((_CI, _D), jnp.float32),
          pltpu.SemaphoreType.DMA,
          pltpu.SemaphoreType.DMA,
          pltpu.SemaphoreType.DMA,
          pltpu.SemaphoreType.DMA,
          pltpu.SemaphoreType.DMA,
          pltpu.SemaphoreType.DMA,
          pltpu.SemaphoreType.DMA,
          pltpu.SemaphoreType.DMA,
          pltpu.SemaphoreType.DMA,
          pltpu.SemaphoreType.DMA,
          pltpu.SemaphoreType.DMA,
      ],
  )
  return f(sample_flat, map_query, map_items_flat, query_table, item_table)


def kernel(sample, map_query, map_pos_items, query_table, item_table):
  out_q, out_i, out_m = _sc_call(
      sample.reshape(-1), map_query, map_pos_items.reshape(-1),
      query_table, item_table)
  return (
      out_q.reshape(_B, _L, _D),
      out_i.reshape(_B, _L, _M, _D),
      out_m.reshape(_B, _L, _M).astype(jnp.bool_),
  )


# speedup vs baseline: 1.2087x; 1.0389x over previous
import jax
import jax.numpy as jnp
from jax import lax
from jax.experimental import pallas as pl
from jax.experimental.pallas import tpu as pltpu
from jax.experimental.pallas import tpu_sc as plsc

_B, _L, _D, _M = 1024, 50, 64, 10
_BL = _B * _L
_NW = 32
_NS = _BL // _NW
_NI = _NS * _M
_CQ = 80
_NQC = _NS // _CQ
_CI = 128
_NIC = _NI // _CI
_PAD = 0


def _q_body(sample_ref, map_query_ref, map_items_flat_ref, query_table_ref,
            out_q_ref, out_m_ref, out_iid_ref,
            sids_v, qids_v, eidx_v, iids_v, mask_v, pad_v,
            qb0, qb1,
            sem_q, sem_i, sem_m, sem_w,
            g0, g1, w0, w1):
  wid = lax.axis_index("s") * 2 + lax.axis_index("c")
  base = wid * _NS
  iota = lax.iota(jnp.int32, 16)
  ten = jnp.full((16,), _M, jnp.int32)
  zrow = jnp.zeros((16,), jnp.float32)
  ones = jnp.full((16,), 1, jnp.int32)
  zeros_i = jnp.full((16,), 0, jnp.int32)
  qbufs = (qb0, qb1)
  gsems = (g0, g1)
  wsems = (w0, w1)

  pltpu.sync_copy(sample_ref.at[pl.ds(base, _NS)], sids_v)

  @pl.loop(0, 12)
  def _qid_fire(k):
    o = k * 128
    pltpu.async_copy(map_query_ref.at[sids_v.at[pl.ds(o, 128)]],
                     qids_v.at[pl.ds(o, 128)], sem_q)
  pltpu.async_copy(map_query_ref.at[sids_v.at[pl.ds(1536, 64)]],
                   qids_v.at[pl.ds(1536, 64)], sem_q)

  pad_v[...] = zeros_i

  @pl.loop(0, _NI // 128)
  def _eidx(k):
    for g in range(8):
      j = k * 128 + g * 16 + iota
      srow = lax.div(j, ten)
      col = j - srow * _M
      sid = plsc.load_gather(sids_v, [srow])
      eidx_v[pl.ds(k * 128 + g * 16, 16)] = sid * _M + col
      pad_v[...] = pad_v[...] | jnp.where(sid == _PAD, ones, zeros_i)

  has_pad = jnp.max(pad_v[...]) != 0

  @pl.loop(0, _NIC)
  def _iid_fire(k):
    o = k * _CI
    pltpu.async_copy(map_items_flat_ref.at[eidx_v.at[pl.ds(o, _CI)]],
                     iids_v.at[pl.ds(o, _CI)], sem_i)

  pltpu.make_async_copy(map_query_ref.at[pl.ds(0, _NS)], qids_v, sem_q).wait()

  pltpu.async_copy(query_table_ref.at[qids_v.at[pl.ds(0, _CQ)]], qb0, g0)
  pltpu.async_copy(query_table_ref.at[qids_v.at[pl.ds(_CQ, _CQ)]], qb1, g1)

  @pl.loop(0, _NQC // 2)
  def _q_ring(k):
    for b in range(2):
      c = k * 2 + b
      o = c * _CQ
      buf, gs, ws = qbufs[b], gsems[b], wsems[b]
      pltpu.make_async_copy(
          query_table_ref.at[qids_v.at[pl.ds(0, _CQ)]], buf, gs).wait()

      @pl.when(has_pad)
      def _fixup():
        for gq in range(_CQ // 16):
          m = sids_v[pl.ds(o + gq * 16, 16)] == _PAD

          @pl.when(jnp.any(m))
          def _zero():
            rows = gq * 16 + iota

            @pl.loop(0, _D)
            def _zc(cc):
              ccol = jnp.full((16,), cc, jnp.int32)
              plsc.store_scatter(buf, [rows, ccol], zrow, mask=m)

      pltpu.async_copy(buf, out_q_ref.at[pl.ds(base + o, _CQ), :], ws)

      @pl.when(c < _NQC - 2)
      def _next():
        pltpu.make_async_copy(
            buf, out_q_ref.at[pl.ds(base, _CQ), :], ws).wait()
        pltpu.async_copy(
            query_table_ref.at[qids_v.at[pl.ds(o + 2 * _CQ, _CQ)]], buf, gs)

  pltpu.make_async_copy(
      map_items_flat_ref.at[pl.ds(0, _NI)], iids_v, sem_i).wait()

  @pl.loop(0, _NI // 16)
  def _mask(k):
    iid = iids_v[pl.ds(k * 16, 16)]
    mask_v[pl.ds(k * 16, 16)] = jnp.where(iid != _PAD, ones, zeros_i)

  @pl.when(has_pad)
  def _mask_fix():
    @pl.loop(0, _NS // 16)
    def _mfix(g):
      sid = sids_v[pl.ds(g * 16, 16)]
      m = sid == _PAD

      @pl.when(jnp.any(m))
      def _zero():
        srow = g * 16 + iota
        for mm in range(_M):
          plsc.store_scatter(mask_v, [srow * _M + mm], zeros_i, mask=m)

  pltpu.async_copy(mask_v, out_m_ref.at[pl.ds(base * _M, _NI)], sem_m)
  pltpu.async_copy(iids_v, out_iid_ref.at[pl.ds(base * _M, _NI)], sem_w)

  pltpu.make_async_copy(qb0, out_q_ref.at[pl.ds(base, _CQ), :], w0).wait()
  pltpu.make_async_copy(qb1, out_q_ref.at[pl.ds(base, _CQ), :], w1).wait()
  pltpu.make_async_copy(
      mask_v, out_m_ref.at[pl.ds(base * _M, _NI)], sem_m).wait()
  pltpu.make_async_copy(
      iids_v, out_iid_ref.at[pl.ds(base * _M, _NI)], sem_w).wait()


def _i_body(sample_ref, iid_ref, item_table_ref,
            out_i_ref,
            sids_v, iids_v, pad_v,
            ib0, ib1, ib2, ib3,
            g0, g1, g2, g3, w0, w1, w2, w3):
  wid = lax.axis_index("s") * 2 + lax.axis_index("c")
  base = wid * _NS
  iota = lax.iota(jnp.int32, 16)
  ten = jnp.full((16,), _M, jnp.int32)
  zrow = jnp.zeros((16,), jnp.float32)
  ones = jnp.full((16,), 1, jnp.int32)
  zeros_i = jnp.full((16,), 0, jnp.int32)
  ibufs = (ib0, ib1, ib2, ib3)
  gsems = (g0, g1, g2, g3)
  wsems = (w0, w1, w2, w3)

  pltpu.sync_copy(sample_ref.at[pl.ds(base, _NS)], sids_v)
  pltpu.sync_copy(iid_ref.at[pl.ds(base * _M, _NI)], iids_v)

  pad_v[...] = zeros_i

  @pl.loop(0, _NS // 16)
  def _pscan(g):
    sid = sids_v[pl.ds(g * 16, 16)]
    pad_v[...] = pad_v[...] | jnp.where(sid == _PAD, ones, zeros_i)

  has_pad = jnp.max(pad_v[...]) != 0

  for b in range(4):
    pltpu.async_copy(
        item_table_ref.at[iids_v.at[pl.ds(b * _CI, _CI)]], ibufs[b], gsems[b])

  @pl.loop(0, (_NIC + 3) // 4)
  def _i_ring(k):
    for b in range(4):
      c = k * 4 + b

      @pl.when(c < _NIC)
      def _chunk():
        o = c * _CI
        buf, gs, ws = ibufs[b], gsems[b], wsems[b]
        pltpu.make_async_copy(
            item_table_ref.at[iids_v.at[pl.ds(0, _CI)]], buf, gs).wait()

        @pl.when(has_pad)
        def _fixup():
          for gq in range(_CI // 16):
            j = o + gq * 16 + iota
            srow = lax.div(j, ten)
            sid = plsc.load_gather(sids_v, [srow])
            m = sid == _PAD

            @pl.when(jnp.any(m))
            def _zero():
              rows = gq * 16 + iota

              @pl.loop(0, _D)
              def _zc(cc):
                ccol = jnp.full((16,), cc, jnp.int32)
                plsc.store_scatter(buf, [rows, ccol], zrow, mask=m)

        pltpu.async_copy(buf, out_i_ref.at[pl.ds(base * _M + o, _CI), :], ws)

        @pl.when(c + 4 < _NIC)
        def _next():
          pltpu.make_async_copy(
              buf, out_i_ref.at[pl.ds(base * _M, _CI), :], ws).wait()
          pltpu.async_copy(
              item_table_ref.at[iids_v.at[pl.ds(o + 4 * _CI, _CI)]], buf, gs)

  for b in range(4):
    pltpu.make_async_copy(
        ibufs[b], out_i_ref.at[pl.ds(base * _M, _CI), :], wsems[b]).wait()


@jax.jit
def _sc_call(sample_flat, map_query, map_items_flat, query_table, item_table):
  mesh = plsc.VectorSubcoreMesh(core_axis_name="c", subcore_axis_name="s")
  params = pltpu.CompilerParams(
      needs_layout_passes=False, use_tc_tiling_on_sc=False)
  fq = pl.kernel(
      _q_body,
      out_type=(
          jax.ShapeDtypeStruct((_BL, _D), jnp.float32),
          jax.ShapeDtypeStruct((_BL * _M,), jnp.int32),
          jax.ShapeDtypeStruct((_BL * _M,), jnp.int32),
      ),
      mesh=mesh,
      compiler_params=params,
      scratch_types=[
          pltpu.VMEM((_NS,), jnp.int32),
          pltpu.VMEM((_NS,), jnp.int32),
          pltpu.VMEM((_NI,), jnp.int32),
          pltpu.VMEM((_NI,), jnp.int32),
          pltpu.VMEM((_NI,), jnp.int32),
          pltpu.VMEM((16,), jnp.int32),
          pltpu.VMEM((_CQ, _D), jnp.float32),
          pltpu.VMEM((_CQ, _D), jnp.float32),
          pltpu.SemaphoreType.DMA,
          pltpu.SemaphoreType.DMA,
          pltpu.SemaphoreType.DMA,
          pltpu.SemaphoreType.DMA,
          pltpu.SemaphoreType.DMA,
          pltpu.SemaphoreType.DMA,
          pltpu.SemaphoreType.DMA,
          pltpu.SemaphoreType.DMA,
      ],
  )
  out_q, out_m, out_iid = fq(sample_flat, map_query, map_items_flat,
                             query_table)

  fi = pl.kernel(
      _i_body,
      out_type=jax.ShapeDtypeStruct((_BL * _M, _D), jnp.float32),
      mesh=mesh,
      compiler_params=params,
      scratch_types=[
          pltpu.VMEM((_NS,), jnp.int32),
          pltpu.VMEM((_NI,), jnp.int32),
          pltpu.VMEM((16,), jnp.int32),
          pltpu.VMEM((_CI, _D), jnp.float32),
          pltpu.VMEM((_CI, _D), jnp.float32),
          pltpu.VMEM((_CI, _D), jnp.float32),
          pltpu.VMEM((_CI, _D), jnp.float32),
          pltpu.SemaphoreType.DMA,
          pltpu.SemaphoreType.DMA,
          pltpu.SemaphoreType.DMA,
          pltpu.SemaphoreType.DMA,
          pltpu.SemaphoreType.DMA,
          pltpu.SemaphoreType.DMA,
          pltpu.SemaphoreType.DMA,
          pltpu.SemaphoreType.DMA,
      ],
  )
  out_i = fi(sample_flat, out_iid, item_table)
  return out_q, out_i, out_m


def kernel(sample, map_query, map_pos_items, query_table, item_table):
  out_q, out_i, out_m = _sc_call(
      sample.reshape(-1), map_query, map_pos_items.reshape(-1),
      query_table, item_table)
  return (
      out_q.reshape(_B, _L, _D),
      out_i.reshape(_B, _L, _M, _D),
      out_m.reshape(_B, _L, _M).astype(jnp.bool_),
  )
